# baseline (speedup 1.0000x reference)
import functools

import jax
import jax.numpy as jnp
from jax import lax
from jax.experimental import pallas as pl
from jax.experimental.pallas import tpu as pltpu
from jax.experimental.pallas import tpu_sc as plsc

B, N, M = 2, 2048, 8192
C_IN, C1, C2 = 64, 64, 64
K = 3
TM = 256
NB = M // TM
BM = B * M

NCORES = 2
NSUB = 16
NW = NCORES * NSUB
QPW = BM // NW
CH = 256


def _knn_body(q_ref, pT_ref, i0_ref, i1_ref, i2_ref, w0_ref, w1_ref, w2_ref):
    b = pl.program_id(0)
    qt = q_ref[0]
    qx, qy, qz = qt[:, 0:1], qt[:, 1:2], qt[:, 2:3]
    pt = pT_ref[0]
    px, py, pz = pt[0:1, :], pt[1:2, :], pt[2:3, :]
    dx = qx - px
    dy = qy - py
    dz = qz - pz
    d = dx * dx + dy * dy + dz * dz
    iota = lax.broadcasted_iota(jnp.int32, (TM, N), 1)

    vals, idxs = [], []
    for k in range(K):
        m = jnp.min(d, axis=1, keepdims=True)
        i = jnp.min(jnp.where(d == m, iota, N), axis=1, keepdims=True)
        vals.append(m)
        idxs.append(i)
        if k < K - 1:
            d = jnp.where(iota == i, jnp.inf, d)

    ws = [1.0 / jnp.maximum(v, 1e-10) for v in vals]
    wsum = ws[0] + ws[1] + ws[2]
    base = b * N
    for i_ref, w_ref, i, w in zip((i0_ref, i1_ref, i2_ref),
                                  (w0_ref, w1_ref, w2_ref), idxs, ws):
        i_ref[...] = i + base
        w_ref[...] = w / wsum


def _sc_gather_body(idx0, idx1, idx2, xT, out0, out1, out2,
                    idx0_v, idx1_v, idx2_v,
                    rows0_v, rows1_v, rows2_v, sem0, sem1, sem2):
    wid = lax.axis_index("s") * NCORES + lax.axis_index("c")
    qb = wid * QPW
    pltpu.sync_copy(idx0.at[pl.ds(qb, QPW)], idx0_v)
    pltpu.sync_copy(idx1.at[pl.ds(qb, QPW)], idx1_v)
    pltpu.sync_copy(idx2.at[pl.ds(qb, QPW)], idx2_v)
    cp0 = pltpu.async_copy(xT.at[idx0_v], rows0_v, sem0)
    cp1 = pltpu.async_copy(xT.at[idx1_v], rows1_v, sem1)
    cp2 = pltpu.async_copy(xT.at[idx2_v], rows2_v, sem2)
    cp0.wait()
    pltpu.sync_copy(rows0_v, out0.at[pl.ds(qb, QPW)])
    cp1.wait()
    pltpu.sync_copy(rows1_v, out1.at[pl.ds(qb, QPW)])
    cp2.wait()
    pltpu.sync_copy(rows2_v, out2.at[pl.ds(qb, QPW)])


TR = 2048
NT = BM // TR


def _accum_stats(u, s_ref, q_ref):
    ps = jnp.sum(u, axis=0, keepdims=True)
    pq = jnp.sum(u * u, axis=0, keepdims=True)

    @pl.when(pl.program_id(0) == 0)
    def _():
        s_ref[...] = jnp.zeros_like(s_ref)
        q_ref[...] = jnp.zeros_like(q_ref)

    s_ref[...] += ps
    q_ref[...] += pq


def _bn_from_stats(u, s_ref, q_ref, g_ref, be_ref):
    mu = s_ref[...] * (1.0 / BM)
    var = q_ref[...] * (1.0 / BM) - mu * mu
    r = (u - mu) * lax.rsqrt(var + 1e-5) * g_ref[...] + be_ref[...]
    return jnp.maximum(r, 0.0)


def _mlp1_body(r0_ref, r1_ref, r2_ref, w0_ref, w1_ref, w2_ref,
               W1_ref, b1_ref, u1_ref, s1_ref, q1_ref):
    h = (r0_ref[...] * w0_ref[...] + r1_ref[...] * w1_ref[...]
         + r2_ref[...] * w2_ref[...])
    u = lax.dot_general(h, W1_ref[...], (((1,), (1,)), ((), ())),
                        preferred_element_type=jnp.float32) + b1_ref[...]
    u1_ref[...] = u
    _accum_stats(u, s1_ref, q1_ref)


def _mlp2_body(u1_ref, s1_ref, q1_ref, g1_ref, be1_ref, W2_ref, b2_ref,
               u2_ref, s2_ref, q2_ref):
    r = _bn_from_stats(u1_ref[...], s1_ref, q1_ref, g1_ref, be1_ref)
    u = lax.dot_general(r, W2_ref[...], (((1,), (1,)), ((), ())),
                        preferred_element_type=jnp.float32) + b2_ref[...]
    u2_ref[...] = u
    _accum_stats(u, s2_ref, q2_ref)


def _mlp3_body(u2_ref, s2_ref, q2_ref, g2_ref, be2_ref, out_ref):
    out_ref[...] = _bn_from_stats(u2_ref[...], s2_ref, q2_ref,
                                  g2_ref, be2_ref)


def _knn_call(q, pT):
    iw_spec = pl.BlockSpec((TM, 1), lambda b, i: (b * NB + i, 0))
    return pl.pallas_call(
        _knn_body,
        grid=(B, NB),
        in_specs=[
            pl.BlockSpec((1, TM, 3), lambda b, i: (b, i, 0)),
            pl.BlockSpec((1, 3, N), lambda b, i: (b, 0, 0)),
        ],
        out_specs=[iw_spec] * 6,
        out_shape=[jax.ShapeDtypeStruct((BM, 1), jnp.int32)] * 3
        + [jax.ShapeDtypeStruct((BM, 1), jnp.float32)] * 3,
    )(q, pT)


@functools.lru_cache(maxsize=1)
def _build_sc_gather():
    @functools.partial(
        pl.kernel,
        out_type=[jax.ShapeDtypeStruct((BM, C_IN), jnp.float32)] * 3,
        mesh=plsc.VectorSubcoreMesh(core_axis_name="c", subcore_axis_name="s"),
        compiler_params=pltpu.CompilerParams(use_tc_tiling_on_sc=False),
        scratch_types=[
            pltpu.VMEM((QPW,), jnp.int32),
            pltpu.VMEM((QPW,), jnp.int32),
            pltpu.VMEM((QPW,), jnp.int32),
            pltpu.VMEM((QPW, C_IN), jnp.float32),
            pltpu.VMEM((QPW, C_IN), jnp.float32),
            pltpu.VMEM((QPW, C_IN), jnp.float32),
            pltpu.SemaphoreType.DMA,
            pltpu.SemaphoreType.DMA,
            pltpu.SemaphoreType.DMA,
        ],
    )
    def _sc_gather(idx0, idx1, idx2, xT, out0, out1, out2, *scratch):
        _sc_gather_body(idx0, idx1, idx2, xT, out0, out1, out2, *scratch)

    return _sc_gather


def kernel(p, q, x, W1, b1, g1, be1, W2, b2, g2, be2):
    pT = jnp.swapaxes(p, 1, 2)
    xTf = jnp.swapaxes(x, 1, 2).reshape(B * N, C_IN)

    i0, i1, i2, w0, w1, w2 = _knn_call(q, pT)
    r0, r1, r2 = _build_sc_gather()(i0.reshape(BM), i1.reshape(BM),
                                    i2.reshape(BM), xTf)

    row = pl.BlockSpec((TR, C1), lambda i: (i, 0))
    w_spec = pl.BlockSpec((TR, 1), lambda i: (i, 0))
    vec = pl.BlockSpec((1, C1), lambda i: (0, 0))
    mat = pl.BlockSpec((C1, C1), lambda i: (0, 0))
    stat_shape = jax.ShapeDtypeStruct((1, C1), jnp.float32)
    row_shape = jax.ShapeDtypeStruct((BM, C1), jnp.float32)

    u1, s1, q1 = pl.pallas_call(
        _mlp1_body, grid=(NT,),
        in_specs=[row, row, row, w_spec, w_spec, w_spec, mat, vec],
        out_specs=[row, vec, vec],
        out_shape=[row_shape, stat_shape, stat_shape],
    )(r0, r1, r2, w0, w1, w2, W1, b1[None, :])

    u2, s2, q2 = pl.pallas_call(
        _mlp2_body, grid=(NT,),
        in_specs=[row, vec, vec, vec, vec, mat, vec],
        out_specs=[row, vec, vec],
        out_shape=[row_shape, stat_shape, stat_shape],
    )(u1, s1, q1, g1[None, :], be1[None, :], W2, b2[None, :])

    out = pl.pallas_call(
        _mlp3_body, grid=(NT,),
        in_specs=[row, vec, vec, vec, vec],
        out_specs=row,
        out_shape=jax.ShapeDtypeStruct((BM, C2), jnp.float32),
    )(u2, s2, q2, g2[None, :], be2[None, :])

    h = jnp.swapaxes(out.reshape(B, M, C2), 1, 2)
    return (q, h)
